# Initial kernel scaffold; baseline (speedup 1.0000x reference)
import jax
import jax.numpy as jnp
from jax.experimental import pallas as pl


def kernel(x, edge_index, W1l, W1r, b1, W2l, W2r, b2):
    raise NotImplementedError("write your pallas kernel here")


# speedup vs baseline: 6.6876x; 6.6876x over previous
import jax
import jax.numpy as jnp
from jax import lax
from jax.experimental import pallas as pl
from jax.experimental.pallas import tpu as pltpu
from jax.experimental.pallas import tpu_sc as plsc
import functools

N = 10000
E = 320000
D = 128

NC = 2
NS = 16
NW = NC * NS
EPW = E // NW
B = 125
NIT = EPW // B
CPI = 10
NCHI = NIT // CPI
RPT = N // NS
NCH = RPT // B

_mesh = plsc.VectorSubcoreMesh(
    core_axis_name="c", subcore_axis_name="s", num_cores=NC, num_subcores=NS
)


def _agg_body(feat, srcr, dstr, znd, out, acc, sidx, didx, rows_v, sem):
    c = lax.axis_index("c")
    s = lax.axis_index("s")
    w = c * NS + s
    r0 = s * RPT

    pltpu.sync_copy(znd, rows_v)
    for k in range(NCH):
        pltpu.sync_copy(rows_v, acc.at[pl.ds(r0 + k * B, B)])
    plsc.subcore_barrier()

    def chunk(ch, carry):
        pltpu.sync_copy(srcr.at[w, ch], sidx)
        pltpu.sync_copy(dstr.at[w, ch], didx)

        def step(j, carry2):
            pltpu.async_copy(feat.at[sidx.at[j]], rows_v, sem).wait()
            pltpu.sync_copy(rows_v, acc.at[didx.at[j]], add=True)
            return carry2

        lax.fori_loop(0, CPI, step, 0)
        return carry

    lax.fori_loop(0, NCHI, chunk, 0)
    plsc.subcore_barrier()

    for k in range(NCH):
        pltpu.sync_copy(acc.at[pl.ds(r0 + k * B, B)], rows_v)
        pltpu.sync_copy(rows_v, out.at[c, s, k])


def _sc_agg(feat, src3, dst3):
    fn = pl.kernel(
        _agg_body,
        out_type=(jax.ShapeDtypeStruct((NC, NS, NCH, B, D), jnp.float32),),
        mesh=_mesh,
        scratch_types=[
            pltpu.VMEM_SHARED((N, D), jnp.float32),
            pltpu.VMEM((CPI, B), jnp.int32),
            pltpu.VMEM((CPI, B), jnp.int32),
            pltpu.VMEM((B, D), jnp.float32),
            pltpu.SemaphoreType.DMA,
        ],
    )
    znd = jnp.zeros((B, D), jnp.float32)
    parts, = fn(feat, src3, dst3, znd)
    return parts.reshape(NC, N, D)


def _cnt_body(dstr, znd, ones_hbm, out, cnt, didx, ones_v, sem):
    c = lax.axis_index("c")
    s = lax.axis_index("s")
    w = c * NS + s
    r0 = s * RPT

    pltpu.sync_copy(znd, ones_v)
    for k in range(NCH):
        pltpu.sync_copy(ones_v, cnt.at[pl.ds(r0 + k * B, B)])
    pltpu.sync_copy(ones_hbm, ones_v)
    plsc.subcore_barrier()

    def chunk(ch, carry):
        pltpu.sync_copy(dstr.at[w, ch], didx)

        def step(j, carry2):
            pltpu.sync_copy(ones_v, cnt.at[didx.at[j]], add=True)
            return carry2

        lax.fori_loop(0, CPI, step, 0)
        return carry

    lax.fori_loop(0, NCHI, chunk, 0)
    plsc.subcore_barrier()

    for k in range(NCH):
        pltpu.sync_copy(cnt.at[pl.ds(r0 + k * B, B)], ones_v)
        pltpu.sync_copy(ones_v, out.at[c, s, k])


def _sc_cnt(dst3):
    fn = pl.kernel(
        _cnt_body,
        out_type=(jax.ShapeDtypeStruct((NC, NS, NCH, B, D), jnp.float32),),
        mesh=_mesh,
        scratch_types=[
            pltpu.VMEM_SHARED((N, D), jnp.float32),
            pltpu.VMEM((CPI, B), jnp.int32),
            pltpu.VMEM((B, D), jnp.float32),
            pltpu.SemaphoreType.DMA,
        ],
    )
    znd = jnp.zeros((B, D), jnp.float32)
    ones_hbm = jnp.ones((B, D), jnp.float32)
    cnts, = fn(dst3, znd, ones_hbm)
    return cnts.reshape(NC, N, D)


def _dense_body(relu, part_ref, cnt_ref, x_ref, wl_ref, wr_ref, b_ref, o_ref):
    part = part_ref[...]
    agg = part[0] + part[1]
    cnt = cnt_ref[...]
    ctot = cnt[0, :, 0:1] + cnt[1, :, 0:1]
    mean = agg / jnp.maximum(ctot, 1.0)
    r = lax.dot_general(mean, wl_ref[...], (((1,), (1,)), ((), ())),
                        preferred_element_type=jnp.float32)
    r = r + lax.dot_general(x_ref[...], wr_ref[...], (((1,), (1,)), ((), ())),
                            preferred_element_type=jnp.float32)
    r = r + b_ref[...]
    if relu:
        r = jnp.maximum(r, 0.0)
    o_ref[...] = r


def _dense(parts, cnts, xin, Wl, Wr, b, relu):
    R = 1000
    return pl.pallas_call(
        functools.partial(_dense_body, relu),
        grid=(N // R,),
        in_specs=[
            pl.BlockSpec((NC, R, D), lambda i: (0, i, 0)),
            pl.BlockSpec((NC, R, D), lambda i: (0, i, 0)),
            pl.BlockSpec((R, D), lambda i: (i, 0)),
            pl.BlockSpec((D, D), lambda i: (0, 0)),
            pl.BlockSpec((D, D), lambda i: (0, 0)),
            pl.BlockSpec((1, D), lambda i: (0, 0)),
        ],
        out_specs=pl.BlockSpec((R, D), lambda i: (i, 0)),
        out_shape=jax.ShapeDtypeStruct((N, D), jnp.float32),
    )(parts, cnts, xin, Wl, Wr, b.reshape(1, D))


def kernel(x, edge_index, W1l, W1r, b1, W2l, W2r, b2):
    src3 = edge_index[0].reshape(NW, NCHI, CPI, B)
    dst3 = edge_index[1].reshape(NW, NCHI, CPI, B)
    cnts = _sc_cnt(dst3)
    parts1 = _sc_agg(x, src3, dst3)
    h = _dense(parts1, cnts, x, W1l, W1r, b1, relu=True)
    parts2 = _sc_agg(h, src3, dst3)
    out = _dense(parts2, cnts, h, W2l, W2r, b2, relu=False)
    return out


# speedup vs baseline: 8.5922x; 1.2848x over previous
import jax
import jax.numpy as jnp
from jax import lax
from jax.experimental import pallas as pl
from jax.experimental.pallas import tpu as pltpu
from jax.experimental.pallas import tpu_sc as plsc
import functools

N = 10000
E = 320000
D = 128

NC = 2
NS = 16
NW = NC * NS
EPW = E // NW
B = 125
NIT = EPW // B
CPI = 10
NCHI = NIT // CPI
RPT = N // NS
NCH = RPT // B

_mesh = plsc.VectorSubcoreMesh(
    core_axis_name="c", subcore_axis_name="s", num_cores=NC, num_subcores=NS
)


def _agg_body(feat, srcr, dstr, znd, out, acc, sidx, didx, rows_v, rows_w,
              sem0, sem1):
    c = lax.axis_index("c")
    s = lax.axis_index("s")
    w = c * NS + s
    r0 = s * RPT

    pltpu.sync_copy(znd, rows_v)
    for k in range(NCH):
        pltpu.sync_copy(rows_v, acc.at[pl.ds(r0 + k * B, B)])
    plsc.subcore_barrier()

    def chunk(ch, carry):
        pltpu.sync_copy(srcr.at[w, ch], sidx)
        pltpu.sync_copy(dstr.at[w, ch], didx)
        pltpu.async_copy(feat.at[sidx.at[0]], rows_v, sem0)

        def pair(p, carry2):
            j0 = 2 * p
            pltpu.async_copy(feat.at[sidx.at[j0 + 1]], rows_w, sem1)
            pltpu.make_async_copy(feat.at[sidx.at[j0]], rows_v, sem0).wait()
            pltpu.sync_copy(rows_v, acc.at[didx.at[j0]], add=True)

            @pl.when(p < CPI // 2 - 1)
            def _():
                pltpu.async_copy(feat.at[sidx.at[j0 + 2]], rows_v, sem0)

            pltpu.make_async_copy(feat.at[sidx.at[j0 + 1]], rows_w, sem1).wait()
            pltpu.sync_copy(rows_w, acc.at[didx.at[j0 + 1]], add=True)
            return carry2

        lax.fori_loop(0, CPI // 2, pair, 0)
        return carry

    lax.fori_loop(0, NCHI, chunk, 0)
    plsc.subcore_barrier()

    for k in range(NCH):
        pltpu.sync_copy(acc.at[pl.ds(r0 + k * B, B)], rows_v)
        pltpu.sync_copy(rows_v, out.at[c, s, k])


def _sc_agg(feat, src3, dst3):
    fn = pl.kernel(
        _agg_body,
        out_type=(jax.ShapeDtypeStruct((NC, NS, NCH, B, D), jnp.float32),),
        mesh=_mesh,
        scratch_types=[
            pltpu.VMEM_SHARED((N, D), jnp.float32),
            pltpu.VMEM((CPI, B), jnp.int32),
            pltpu.VMEM((CPI, B), jnp.int32),
            pltpu.VMEM((B, D), jnp.float32),
            pltpu.VMEM((B, D), jnp.float32),
            pltpu.SemaphoreType.DMA,
            pltpu.SemaphoreType.DMA,
        ],
    )
    znd = jnp.zeros((B, D), jnp.float32)
    parts, = fn(feat, src3, dst3, znd)
    return parts.reshape(NC, N, D)


def _cnt_body(dstr, znd, ones_hbm, out, cnt, didx, ones_v, sem):
    c = lax.axis_index("c")
    s = lax.axis_index("s")
    w = c * NS + s
    r0 = s * RPT

    pltpu.sync_copy(znd, ones_v)
    for k in range(NCH):
        pltpu.sync_copy(ones_v, cnt.at[pl.ds(r0 + k * B, B)])
    pltpu.sync_copy(ones_hbm, ones_v)
    plsc.subcore_barrier()

    def chunk(ch, carry):
        pltpu.sync_copy(dstr.at[w, ch], didx)

        def fire(j, carry2):
            pltpu.async_copy(ones_v, cnt.at[didx.at[j]], sem, add=True)
            return carry2

        lax.fori_loop(0, CPI, fire, 0)

        def drain(j, carry2):
            pltpu.make_async_copy(ones_v, cnt.at[didx.at[j]], sem).wait()
            return carry2

        lax.fori_loop(0, CPI, drain, 0)
        return carry

    lax.fori_loop(0, NCHI, chunk, 0)
    plsc.subcore_barrier()

    for k in range(NCH):
        pltpu.sync_copy(cnt.at[pl.ds(r0 + k * B, B)], ones_v)
        pltpu.sync_copy(ones_v, out.at[c, s, k])


def _sc_cnt(dst3):
    fn = pl.kernel(
        _cnt_body,
        out_type=(jax.ShapeDtypeStruct((NC, NS, NCH, B, D), jnp.float32),),
        mesh=_mesh,
        scratch_types=[
            pltpu.VMEM_SHARED((N, D), jnp.float32),
            pltpu.VMEM((CPI, B), jnp.int32),
            pltpu.VMEM((B, D), jnp.float32),
            pltpu.SemaphoreType.DMA,
        ],
    )
    znd = jnp.zeros((B, D), jnp.float32)
    ones_hbm = jnp.ones((B, D), jnp.float32)
    cnts, = fn(dst3, znd, ones_hbm)
    return cnts.reshape(NC, N, D)


def _dense_body(relu, part_ref, cnt_ref, x_ref, wl_ref, wr_ref, b_ref, o_ref):
    part = part_ref[...]
    agg = part[0] + part[1]
    cnt = cnt_ref[...]
    ctot = cnt[0, :, 0:1] + cnt[1, :, 0:1]
    mean = agg / jnp.maximum(ctot, 1.0)
    r = lax.dot_general(mean, wl_ref[...], (((1,), (1,)), ((), ())),
                        preferred_element_type=jnp.float32)
    r = r + lax.dot_general(x_ref[...], wr_ref[...], (((1,), (1,)), ((), ())),
                            preferred_element_type=jnp.float32)
    r = r + b_ref[...]
    if relu:
        r = jnp.maximum(r, 0.0)
    o_ref[...] = r


def _dense(parts, cnts, xin, Wl, Wr, b, relu):
    R = 1000
    return pl.pallas_call(
        functools.partial(_dense_body, relu),
        grid=(N // R,),
        in_specs=[
            pl.BlockSpec((NC, R, D), lambda i: (0, i, 0)),
            pl.BlockSpec((NC, R, D), lambda i: (0, i, 0)),
            pl.BlockSpec((R, D), lambda i: (i, 0)),
            pl.BlockSpec((D, D), lambda i: (0, 0)),
            pl.BlockSpec((D, D), lambda i: (0, 0)),
            pl.BlockSpec((1, D), lambda i: (0, 0)),
        ],
        out_specs=pl.BlockSpec((R, D), lambda i: (i, 0)),
        out_shape=jax.ShapeDtypeStruct((N, D), jnp.float32),
    )(parts, cnts, xin, Wl, Wr, b.reshape(1, D))


def kernel(x, edge_index, W1l, W1r, b1, W2l, W2r, b2):
    src3 = edge_index[0].reshape(NW, NCHI, CPI, B)
    dst3 = edge_index[1].reshape(NW, NCHI, CPI, B)
    cnts = _sc_cnt(dst3)
    parts1 = _sc_agg(x, src3, dst3)
    h = _dense(parts1, cnts, x, W1l, W1r, b1, relu=True)
    parts2 = _sc_agg(h, src3, dst3)
    out = _dense(parts2, cnts, h, W2l, W2r, b2, relu=False)
    return out


# speedup vs baseline: 9.6175x; 1.1193x over previous
import jax
import jax.numpy as jnp
from jax import lax
from jax.experimental import pallas as pl
from jax.experimental.pallas import tpu as pltpu
from jax.experimental.pallas import tpu_sc as plsc
import functools

N = 10000
E = 320000
D = 128

NC = 2
NS = 16
NW = NC * NS
EPW = E // NW
B = 125
NIT = EPW // B
CPI = 10
NCHI = NIT // CPI
RPT = N // NS
NCH = RPT // B

BA = 80
CPA = 25
NCHA = EPW // (BA * CPA)

_mesh = plsc.VectorSubcoreMesh(
    core_axis_name="c", subcore_axis_name="s", num_cores=NC, num_subcores=NS
)


NZCH = N // BA


def _agg_body(feat, srcr, dstr, znd, out,
              acc, sidx, didx, r0b, r1b, r2b, r3b, semg, sems):
    c = lax.axis_index("c")
    s = lax.axis_index("s")
    w = c * NS + s
    rows = [r0b, r1b, r2b, r3b]

    pltpu.sync_copy(znd, r0b)
    for m in range(NZCH // NS + 1):
        cid = s + m * NS

        @pl.when(cid < NZCH)
        def _():
            pltpu.sync_copy(r0b, acc.at[pl.ds(cid * BA, BA)])

    plsc.subcore_barrier()

    def chunk(ch, carry):
        pltpu.sync_copy(srcr.at[w, ch], sidx)
        pltpu.sync_copy(dstr.at[w, ch], didx)
        pltpu.async_copy(feat.at[sidx.at[0]], rows[0], semg)
        pltpu.async_copy(feat.at[sidx.at[1]], rows[1], semg)
        for j in range(CPA):
            bj = rows[j % 4]
            pltpu.make_async_copy(feat.at[sidx.at[j]], bj, semg).wait()
            pltpu.async_copy(bj, acc.at[didx.at[j]], sems, add=True)
            if j >= 2:
                bp = rows[(j - 2) % 4]
                pltpu.make_async_copy(bp, acc.at[didx.at[j - 2]], sems).wait()
            if j + 2 < CPA:
                pltpu.async_copy(feat.at[sidx.at[j + 2]], rows[(j + 2) % 4], semg)
        for j in (CPA - 2, CPA - 1):
            pltpu.make_async_copy(rows[j % 4], acc.at[didx.at[j]], sems).wait()
        return carry

    lax.fori_loop(0, NCHA, chunk, 0)
    plsc.subcore_barrier()

    for m in range(NZCH // NS + 1):
        cid = s + m * NS

        @pl.when(cid < NZCH)
        def _():
            pltpu.sync_copy(acc.at[pl.ds(cid * BA, BA)], r0b)
            pltpu.sync_copy(r0b, out.at[c, cid])


def _sc_agg(feat, src3, dst3):
    fn = pl.kernel(
        _agg_body,
        out_type=(jax.ShapeDtypeStruct((NC, NZCH, BA, D), jnp.float32),),
        mesh=_mesh,
        scratch_types=[
            pltpu.VMEM_SHARED((N, D), jnp.float32),
            pltpu.VMEM((CPA, BA), jnp.int32),
            pltpu.VMEM((CPA, BA), jnp.int32),
            pltpu.VMEM((BA, D), jnp.float32),
            pltpu.VMEM((BA, D), jnp.float32),
            pltpu.VMEM((BA, D), jnp.float32),
            pltpu.VMEM((BA, D), jnp.float32),
            pltpu.SemaphoreType.DMA,
            pltpu.SemaphoreType.DMA,
        ],
    )
    znd = jnp.zeros((BA, D), jnp.float32)
    parts, = fn(feat, src3, dst3, znd)
    return parts.reshape(NC, N, D)


def _cnt_body(dstr, znd, ones_hbm, out, cnt, didx, ones_v, sem):
    c = lax.axis_index("c")
    s = lax.axis_index("s")
    w = c * NS + s
    r0 = s * RPT

    pltpu.sync_copy(znd, ones_v)
    for k in range(NCH):
        pltpu.sync_copy(ones_v, cnt.at[pl.ds(r0 + k * B, B)])
    pltpu.sync_copy(ones_hbm, ones_v)
    plsc.subcore_barrier()

    def chunk(ch, carry):
        pltpu.sync_copy(dstr.at[w, ch], didx)

        def fire(j, carry2):
            pltpu.async_copy(ones_v, cnt.at[didx.at[j]], sem, add=True)
            return carry2

        lax.fori_loop(0, CPI, fire, 0)

        def drain(j, carry2):
            pltpu.make_async_copy(ones_v, cnt.at[didx.at[j]], sem).wait()
            return carry2

        lax.fori_loop(0, CPI, drain, 0)
        return carry

    lax.fori_loop(0, NCHI, chunk, 0)
    plsc.subcore_barrier()

    for k in range(NCH):
        pltpu.sync_copy(cnt.at[pl.ds(r0 + k * B, B)], ones_v)
        pltpu.sync_copy(ones_v, out.at[c, s, k])


def _sc_cnt(dst3):
    fn = pl.kernel(
        _cnt_body,
        out_type=(jax.ShapeDtypeStruct((NC, NS, NCH, B, D), jnp.float32),),
        mesh=_mesh,
        scratch_types=[
            pltpu.VMEM_SHARED((N, D), jnp.float32),
            pltpu.VMEM((CPI, B), jnp.int32),
            pltpu.VMEM((B, D), jnp.float32),
            pltpu.SemaphoreType.DMA,
        ],
    )
    znd = jnp.zeros((B, D), jnp.float32)
    ones_hbm = jnp.ones((B, D), jnp.float32)
    cnts, = fn(dst3, znd, ones_hbm)
    return cnts.reshape(NC, N, D)


def _dense_body(relu, part_ref, cnt_ref, x_ref, wl_ref, wr_ref, b_ref, o_ref):
    part = part_ref[...]
    agg = part[0] + part[1]
    cnt = cnt_ref[...]
    ctot = cnt[0, :, 0:1] + cnt[1, :, 0:1]
    mean = agg / jnp.maximum(ctot, 1.0)
    r = lax.dot_general(mean, wl_ref[...], (((1,), (1,)), ((), ())),
                        preferred_element_type=jnp.float32)
    r = r + lax.dot_general(x_ref[...], wr_ref[...], (((1,), (1,)), ((), ())),
                            preferred_element_type=jnp.float32)
    r = r + b_ref[...]
    if relu:
        r = jnp.maximum(r, 0.0)
    o_ref[...] = r


def _dense(parts, cnts, xin, Wl, Wr, b, relu):
    R = 1000
    return pl.pallas_call(
        functools.partial(_dense_body, relu),
        grid=(N // R,),
        in_specs=[
            pl.BlockSpec((NC, R, D), lambda i: (0, i, 0)),
            pl.BlockSpec((NC, R, D), lambda i: (0, i, 0)),
            pl.BlockSpec((R, D), lambda i: (i, 0)),
            pl.BlockSpec((D, D), lambda i: (0, 0)),
            pl.BlockSpec((D, D), lambda i: (0, 0)),
            pl.BlockSpec((1, D), lambda i: (0, 0)),
        ],
        out_specs=pl.BlockSpec((R, D), lambda i: (i, 0)),
        out_shape=jax.ShapeDtypeStruct((N, D), jnp.float32),
    )(parts, cnts, xin, Wl, Wr, b.reshape(1, D))


def kernel(x, edge_index, W1l, W1r, b1, W2l, W2r, b2):
    src3a = edge_index[0].reshape(NW, NCHA, CPA, BA)
    dst3a = edge_index[1].reshape(NW, NCHA, CPA, BA)
    dst3c = edge_index[1].reshape(NW, NCHI, CPI, B)
    cnts = _sc_cnt(dst3c)
    parts1 = _sc_agg(x, src3a, dst3a)
    h = _dense(parts1, cnts, x, W1l, W1r, b1, relu=True)
    parts2 = _sc_agg(h, src3a, dst3a)
    out = _dense(parts2, cnts, h, W2l, W2r, b2, relu=False)
    return out
